# baseline (speedup 1.0000x reference)
import functools

import jax
import jax.numpy as jnp
from jax import lax
from jax.experimental import pallas as pl
from jax.experimental.pallas import tpu as pltpu
from jax.experimental.pallas import tpu_sc as plsc

_D = 64
_E = 8
_ROWS = 1024 * 200

_NC = 2
_NS = 16
_NW = _NC * _NS
_PER_W = _ROWS // _NW
_CHUNK = 256
_NCHUNK = _PER_W // _CHUNK


def _body(wid_hbm, en_hbm, table_hbm, out_hbm,
          idx_v, eni_v, out_v, drain_v, sem_r):
    w = lax.axis_index("s") * _NC + lax.axis_index("c")
    tile_base = w * _PER_W
    lane = lax.iota(jnp.int32, 16)

    def chunk(ci, carry):
        base = pl.multiple_of(tile_base + ci * _CHUNK, 8)
        pltpu.sync_copy(wid_hbm.at[pl.ds(base, _CHUNK)], idx_v)
        pltpu.sync_copy(en_hbm.at[pl.ds(base, _CHUNK)], eni_v)

        def fire(gi, carry2):
            g0 = gi * 16
            ivec = idx_v[pl.ds(g0, 16)]
            for u in range(16):
                pltpu.async_copy(
                    table_hbm.at[ivec[u]],
                    out_v.at[g0 + u, pl.ds(0, _D)],
                    sem_r)
            return carry2

        lax.fori_loop(0, _CHUNK // 16, fire, 0)
        pltpu.make_async_copy(
            wid_hbm.at[pl.ds(0, _CHUNK * _D)],
            drain_v,
            sem_r).wait()

        def blend(gi, carry2):
            g0 = gi * 16
            ev = eni_v[pl.ds(g0, 16)].astype(jnp.float32)
            for u in range(16):
                r = g0 + u
                tail = out_v[r, pl.ds(_D - 8, 16)]
                out_v[r, pl.ds(_D - 8, 16)] = jnp.where(lane < 8, tail, ev[u])
            return carry2

        lax.fori_loop(0, _CHUNK // 16, blend, 0)
        pltpu.sync_copy(out_v, out_hbm.at[pl.ds(base, _CHUNK)])
        return carry

    lax.fori_loop(0, _NCHUNK, chunk, 0)


@jax.jit
def _run(wid, en, table):
    mesh = plsc.VectorSubcoreMesh(core_axis_name="c", subcore_axis_name="s")
    f = functools.partial(
        pl.kernel,
        mesh=mesh,
        out_type=jax.ShapeDtypeStruct((_ROWS, _D + _E), jnp.float32),
        scratch_types=[
            pltpu.VMEM((_CHUNK,), jnp.int32),
            pltpu.VMEM((_CHUNK,), jnp.int32),
            pltpu.VMEM((_CHUNK, _D + _E), jnp.float32),
            pltpu.VMEM((_CHUNK * _D,), jnp.int32),
            pltpu.SemaphoreType.DMA,
        ],
    )(_body)
    return f(wid, en, table)


def kernel(word_id, en_indicator, table):
    b, s = word_id.shape
    wid = word_id.reshape(-1)
    en = en_indicator.reshape(-1)
    out = _run(wid, en, table)
    return out.reshape(b, s, _D + _E)
